# baseline (speedup 1.0000x reference)
import jax
import jax.numpy as jnp
from jax import lax
from jax.experimental import pallas as pl
from jax.experimental.pallas import tpu as pltpu
from jax.experimental.pallas import tpu_sc as plsc

BATCH = 16384
LATENT_DIM = 128
NUM_CLASS = 1000

_NC = 2
_NS = 16
_NW = _NC * _NS
_LANES = 16

_B_PER_W = BATCH // _NW
_CH = 128
_NCHUNK = _B_PER_W // _CH
_VPR = LATENT_DIM // _LANES


_STAGERS = 8
_ROWS_PER_STAGER = 120
_ROWS_LAST = NUM_CLASS - _STAGERS * _ROWS_PER_STAGER


def _body(table_hbm, idx_hbm, z_hbm, out_hbm, idx_v, table_sh,
          rows0, rows1, zfull, gs0, gs1, zs0, os0, os1):
    sid = lax.axis_index("s")
    wid = sid * _NC + lax.axis_index("c")
    base = wid * _B_PER_W

    rows = (rows0, rows1)
    gsem = (gs0, gs1)
    osem = (os0, os1)

    zd = pltpu.async_copy(z_hbm.at[pl.ds(base, _B_PER_W)], zfull, zs0)

    with jax.named_scope("ph_idx"):
        pltpu.sync_copy(idx_hbm.at[pl.ds(base, _B_PER_W)], idx_v)

    gd = [None, None]
    od = [None, None]

    def start(k):
        p = k % 2
        src = table_hbm if k == 0 else table_sh
        gd[p] = pltpu.async_copy(
            src.at[idx_v.at[pl.ds(k * _CH, _CH)]], rows[p], gsem[p])

    start(0)

    with jax.named_scope("ph_stage"):
        @pl.when(sid < _STAGERS)
        def _stage():
            pltpu.sync_copy(
                table_hbm.at[pl.ds(sid * _ROWS_PER_STAGER, _ROWS_PER_STAGER)],
                table_sh.at[pl.ds(sid * _ROWS_PER_STAGER, _ROWS_PER_STAGER)])

        @pl.when(sid == _STAGERS)
        def _stage_last():
            pltpu.sync_copy(
                table_hbm.at[pl.ds(_STAGERS * _ROWS_PER_STAGER, _ROWS_LAST)],
                table_sh.at[pl.ds(_STAGERS * _ROWS_PER_STAGER, _ROWS_LAST)])

    with jax.named_scope("ph_barrier"):
        plsc.subcore_barrier()
    for k in range(_NCHUNK):
        p = k % 2
        q = (k + 1) % 2
        if k + 1 < _NCHUNK:
            if od[q] is not None:
                od[q].wait()
            start(k + 1)
        with jax.named_scope(f"ph_wg{k}"):
            gd[p].wait()
        if k == 0:
            with jax.named_scope("ph_wz"):
                zd.wait()

        zoff = k * _CH

        with jax.named_scope(f"ph_mul{k}"):
            @plsc.parallel_loop(0, _CH, unroll=16)
            def _mul_row(r):
                for c in range(_VPR):
                    sl = pl.ds(c * _LANES, _LANES)
                    rows[p][r, sl] = rows[p][r, sl] * zfull[zoff + r, sl]
        od[p] = pltpu.async_copy(
            rows[p], out_hbm.at[pl.ds(base + k * _CH, _CH)], osem[p])

    with jax.named_scope("ph_tail"):
        od[(_NCHUNK - 2) % 2].wait()
        od[(_NCHUNK - 1) % 2].wait()


@jax.jit
def _run(table, label_i32, z):
    mesh = plsc.VectorSubcoreMesh(core_axis_name="c", subcore_axis_name="s")
    buf = pltpu.VMEM((_CH, LATENT_DIM), jnp.float32)
    return pl.kernel(
        _body,
        out_type=jax.ShapeDtypeStruct((BATCH, LATENT_DIM), jnp.float32),
        mesh=mesh,
        scratch_types=[
            pltpu.VMEM((_B_PER_W,), jnp.int32),
            pltpu.VMEM_SHARED((NUM_CLASS, LATENT_DIM), jnp.float32),
            buf, buf,
            pltpu.VMEM((_B_PER_W, LATENT_DIM), jnp.float32),
            pltpu.SemaphoreType.DMA, pltpu.SemaphoreType.DMA,
            pltpu.SemaphoreType.DMA,
            pltpu.SemaphoreType.DMA, pltpu.SemaphoreType.DMA,
        ],
    )(table, label_i32, z)


def kernel(z, label, emb_table):
    return _run(emb_table, label.astype(jnp.int32), z)


# speedup vs baseline: 1.1705x; 1.1705x over previous
import jax
import jax.numpy as jnp
from jax import lax
from jax.experimental import pallas as pl
from jax.experimental.pallas import tpu as pltpu
from jax.experimental.pallas import tpu_sc as plsc

BATCH = 16384
LATENT_DIM = 128
NUM_CLASS = 1000

_NC = 2
_NS = 16
_NW = _NC * _NS
_LANES = 16

_B_PER_W = BATCH // _NW
_CH = 128
_NCHUNK = _B_PER_W // _CH
_VPR = LATENT_DIM // _LANES

_STAGERS = 5
_ROWS_PER_STAGER = NUM_CLASS // _STAGERS


def _body(table_hbm, idx_hbm, z_hbm, out_hbm, idx_v, table_sh,
          rows0, rows1, zb0, zb1, gs0, gs1, zs0, zs1, os0, os1):
    sid = lax.axis_index("s")
    wid = sid * _NC + lax.axis_index("c")
    base = wid * _B_PER_W

    rows = (rows0, rows1)
    zb = (zb0, zb1)
    gsem = (gs0, gs1)
    zsem = (zs0, zs1)
    osem = (os0, os1)

    gd = [None, None]
    zd = [None, None]
    od = [None, None]

    with jax.named_scope("ph_idx0"):
        pltpu.sync_copy(idx_hbm.at[pl.ds(base, _CH)],
                        idx_v.at[pl.ds(0, _CH)])

    zd[0] = pltpu.async_copy(z_hbm.at[pl.ds(base, _CH)], zb[0], zsem[0])
    gd[0] = pltpu.async_copy(
        table_hbm.at[idx_v.at[pl.ds(0, _CH)]], rows[0], gsem[0])

    with jax.named_scope("ph_idx1"):
        pltpu.sync_copy(idx_hbm.at[pl.ds(base + _CH, _B_PER_W - _CH)],
                        idx_v.at[pl.ds(_CH, _B_PER_W - _CH)])

    with jax.named_scope("ph_stage"):
        @pl.when(sid < _STAGERS)
        def _stage():
            pltpu.sync_copy(
                table_hbm.at[pl.ds(sid * _ROWS_PER_STAGER, _ROWS_PER_STAGER)],
                table_sh.at[pl.ds(sid * _ROWS_PER_STAGER, _ROWS_PER_STAGER)])

    with jax.named_scope("ph_barrier"):
        plsc.subcore_barrier()

    def start(k):
        p = k % 2
        gd[p] = pltpu.async_copy(
            table_sh.at[idx_v.at[pl.ds(k * _CH, _CH)]], rows[p], gsem[p])
        zd[p] = pltpu.async_copy(
            z_hbm.at[pl.ds(base + k * _CH, _CH)], zb[p], zsem[p])

    for k in range(_NCHUNK):
        p = k % 2
        q = (k + 1) % 2
        if k + 1 < _NCHUNK:
            if od[q] is not None:
                od[q].wait()
            start(k + 1)
        with jax.named_scope(f"ph_wg{k}"):
            gd[p].wait()
        with jax.named_scope(f"ph_wz{k}"):
            zd[p].wait()

        with jax.named_scope(f"ph_mul{k}"):
            @plsc.parallel_loop(0, _CH, unroll=8)
            def _mul_row(r):
                for c in range(_VPR):
                    sl = pl.ds(c * _LANES, _LANES)
                    rows[p][r, sl] = rows[p][r, sl] * zb[p][r, sl]
        od[p] = pltpu.async_copy(
            rows[p], out_hbm.at[pl.ds(base + k * _CH, _CH)], osem[p])

    with jax.named_scope("ph_tail"):
        od[(_NCHUNK - 2) % 2].wait()
        od[(_NCHUNK - 1) % 2].wait()


@jax.jit
def _run(table, label_i32, z):
    mesh = plsc.VectorSubcoreMesh(core_axis_name="c", subcore_axis_name="s")
    buf = pltpu.VMEM((_CH, LATENT_DIM), jnp.float32)
    return pl.kernel(
        _body,
        out_type=jax.ShapeDtypeStruct((BATCH, LATENT_DIM), jnp.float32),
        mesh=mesh,
        scratch_types=[
            pltpu.VMEM((_B_PER_W,), jnp.int32),
            pltpu.VMEM_SHARED((NUM_CLASS, LATENT_DIM), jnp.float32),
            buf, buf, buf, buf,
            pltpu.SemaphoreType.DMA, pltpu.SemaphoreType.DMA,
            pltpu.SemaphoreType.DMA, pltpu.SemaphoreType.DMA,
            pltpu.SemaphoreType.DMA, pltpu.SemaphoreType.DMA,
        ],
    )(table, label_i32, z)


def kernel(z, label, emb_table):
    return _run(emb_table, label.astype(jnp.int32), z)
